# baseline (speedup 1.0000x reference)
import jax
import jax.numpy as jnp
from jax.experimental import pallas as pl
from jax.experimental.pallas import tpu as pltpu
from jax.experimental.pallas import tpu_sc as plsc

_NH = 16
_NB = 32
_Q = 2048
_K = 2048
_LINE = 2 * _Q
_THR = (8, 12, 16, 23, 32, 46, 64, 91)
_NWORK = 32
_ROWS = _Q // _NWORK
_RING = 4


_WPAD = 4224
_WSH = 4160
_NGRP = _WSH // 8


def _line_kernel(table_ref, line_ref):
    j = jax.lax.broadcasted_iota(jnp.int32, (1, _WPAD), 1)
    d = j - (_Q - 1)
    a = jnp.abs(d)
    v = jnp.minimum(a, 7)
    for t in _THR:
        v = v + (a >= t).astype(jnp.int32)
    bucket = jnp.where(d > 0, 16, 0) + v
    acc = jnp.zeros((_NH, _WPAD), jnp.float32)
    for b in range(_NB):
        col = table_ref[b, :].reshape(_NH, 1)
        acc = jnp.where(bucket == b, col, acc)
    for s in range(8):
        line_ref[s] = acc[:, s:s + _WSH]


_NHW = 2
_QSL = 512
_BLK = 8
_NBLK = _QSL // _BLK


def _sc_expand(line8):
    @pl.kernel(
        out_type=jax.ShapeDtypeStruct((_NH, _Q, _K // 8, 8), jnp.float32),
        mesh=plsc.VectorSubcoreMesh(core_axis_name="c", subcore_axis_name="s"),
        scratch_types=[
            pltpu.VMEM((2, _NHW, _BLK, _K // 8, 8), jnp.float32),
            pltpu.SemaphoreType.DMA((2, _BLK)),
            pltpu.SemaphoreType.DMA((2,)),
        ],
        compiler_params=pltpu.CompilerParams(use_tc_tiling_on_sc=False),
    )
    def body(line_hbm, out_hbm, st_ref, sst, sout):
        w = jax.lax.axis_index("c") * 16 + jax.lax.axis_index("s")
        h0 = jax.lax.div(w, 4) * _NHW
        q0 = jax.lax.rem(w, 4) * _QSL

        def stage_cp(buf, i, q):
            start = (_Q - 1) - q
            s = jax.lax.rem(start, 8)
            c = jax.lax.div(start, 8)
            return pltpu.make_async_copy(
                line_hbm.at[s, pl.ds(h0, _NHW), pl.ds(c, _K // 8), :],
                st_ref.at[buf, :, i, :, :],
                sst.at[buf, i],
            )

        def out_cp(buf, qb):
            return pltpu.make_async_copy(
                st_ref.at[buf],
                out_hbm.at[pl.ds(h0, _NHW), pl.ds(qb, _BLK), :, :],
                sout.at[buf],
            )

        @pl.loop(0, _NBLK)
        def _(b):
            buf = jax.lax.rem(b, 2)
            qb = q0 + b * _BLK

            @pl.when(b >= 2)
            def _():
                out_cp(buf, qb - 2 * _BLK).wait()

            for i in range(_BLK):
                stage_cp(buf, i, qb + i).start()
            for i in range(_BLK):
                stage_cp(buf, i, qb + i).wait()
            out_cp(buf, qb).start()

        for b in (_NBLK - 2, _NBLK - 1):
            out_cp(b % 2, q0 + b * _BLK).wait()

    return body(line8)


def kernel(x, table):
    del x
    line8 = pl.pallas_call(
        _line_kernel,
        out_shape=jax.ShapeDtypeStruct((8, _NH, _WSH), jnp.float32),
    )(table)
    line8 = line8.reshape(8, _NH, _NGRP, 8)
    return _sc_expand(line8).reshape(_NH, _Q, _K)


_ = _NWORK, _ROWS, _RING, _LINE


# speedup vs baseline: 15.9879x; 15.9879x over previous
import jax
import jax.numpy as jnp
from jax.experimental import pallas as pl
from jax.experimental.pallas import tpu as pltpu
from jax.experimental.pallas import tpu_sc as plsc

_NH = 16
_NB = 32
_Q = 2048
_K = 2048
_THR = (8, 12, 16, 23, 32, 46, 64, 91)

_NP = 136
_LAW = 4096
_EXT = 4352
_NHW = 2
_QSL = 512
_BLK = 8
_NBLK = _QSL // _BLK


def _la_kernel(table_ref, la_ref, lext_ref, u_ref, sems):
    p = pl.program_id(0)
    par = jax.lax.rem(p, 2)

    @pl.when(p == 0)
    def _():
        j = jax.lax.broadcasted_iota(jnp.int32, (1, _EXT), 1)
        d = j - 128 - (_Q - 1)
        a = jnp.abs(d)
        v = jnp.minimum(a, 7)
        for t in _THR:
            v = v + (a >= t).astype(jnp.int32)
        bucket = jnp.where(d > 0, 16, 0) + v
        acc = jnp.zeros((_NH, _EXT), jnp.float32)
        for b in range(_NB):
            col = table_ref[b, :].reshape(_NH, 1)
            acc = jnp.where(bucket == b, col, acc)
        lext_ref[...] = acc

    def cp(par_, p_):
        return pltpu.make_async_copy(
            u_ref.at[par_, :, pl.ds(0, _LAW)],
            la_ref.at[:, p_, :],
            sems.at[par_],
        )

    @pl.when(p >= 2)
    def _():
        cp(par, p - 2).wait()

    shift = jax.lax.rem(jnp.int32(_EXT - 135) + p, jnp.int32(_EXT))
    u_ref[par] = pltpu.roll(lext_ref[...], shift, 1)
    cp(par, p).start()

    @pl.when(p == _NP - 1)
    def _():
        cp(1 - par, p - 1).wait()
        cp(par, p).wait()


def _build_la(table):
    return pl.pallas_call(
        _la_kernel,
        grid=(_NP,),
        in_specs=[pl.BlockSpec((_NB, _NH), lambda p: (0, 0))],
        out_specs=pl.BlockSpec(memory_space=pl.ANY),
        out_shape=jax.ShapeDtypeStruct((_NH, _NP, _LAW), jnp.float32),
        scratch_shapes=[
            pltpu.VMEM((_NH, _EXT), jnp.float32),
            pltpu.VMEM((2, _NH, _EXT), jnp.float32),
            pltpu.SemaphoreType.DMA((2,)),
        ],
    )(table)


def _sc_expand(la):
    @pl.kernel(
        out_type=jax.ShapeDtypeStruct((_NH, _Q, _K), jnp.float32),
        mesh=plsc.VectorSubcoreMesh(core_axis_name="c", subcore_axis_name="s"),
        scratch_types=[
            pltpu.VMEM((2, _NHW, _BLK, _K), jnp.float32),
            pltpu.SemaphoreType.DMA((2,)),
            pltpu.SemaphoreType.DMA((2,)),
        ],
        compiler_params=pltpu.CompilerParams(use_tc_tiling_on_sc=True),
    )
    def body(la_hbm, out_hbm, st_ref, sst, sout):
        w = jax.lax.axis_index("c") * 16 + jax.lax.axis_index("s")
        h0 = jax.lax.div(w, 4) * _NHW
        q0 = jax.lax.rem(w, 4) * _QSL

        def stage_cp(buf, qb):
            s0 = (_Q - 1) - qb
            c2 = jax.lax.div(s0 + 120, 128)
            p0x8 = jax.lax.div(c2 * 128 + 7 - s0, 8) * 8
            return pltpu.make_async_copy(
                la_hbm.at[pl.ds(h0, _NHW), pl.ds(p0x8, _BLK),
                          pl.ds(c2 * 128, _K)],
                st_ref.at[buf],
                sst.at[buf],
            )

        def out_cp(buf, qb):
            return pltpu.make_async_copy(
                st_ref.at[buf],
                out_hbm.at[pl.ds(h0, _NHW), pl.ds(qb, _BLK), :],
                sout.at[buf],
            )

        @pl.loop(0, _NBLK)
        def _(b):
            buf = jax.lax.rem(b, 2)
            qb = (jax.lax.div(q0, _BLK) + b) * _BLK

            @pl.when(b >= 2)
            def _():
                out_cp(buf, qb - 2 * _BLK).wait()

            stage_cp(buf, qb).start()
            stage_cp(buf, qb).wait()
            out_cp(buf, qb).start()

        for b in (_NBLK - 2, _NBLK - 1):
            out_cp(b % 2, q0 + b * _BLK).wait()

    return body(la)


def kernel(x, table):
    del x
    return _sc_expand(_build_la(table))
